# Initial kernel scaffold; baseline (speedup 1.0000x reference)
import jax
import jax.numpy as jnp
from jax.experimental import pallas as pl


def kernel(edge_distance_vec, x_edge, edge_index, batch, W1, b1, W2, b2):
    raise NotImplementedError("write your pallas kernel here")


# speedup vs baseline: 23.7932x; 23.7932x over previous
import jax
import jax.numpy as jnp
from jax.experimental import pallas as pl
from jax.experimental.pallas import tpu as pltpu

_N_GRAPHS = 16


def _pick_block(total, candidates):
    for c in candidates:
        if total % c == 0:
            return c
    return total


def _r2_kernel(idx_ref, x_ref, edv_ref, bat_ref, W1_ref, b1_ref, W2t_ref,
               b2_ref, out_ref, acc_ref, pref_ref, node_ref,
               gacc_ref, *, n_edge_blocks, n_node_blocks, be, bn, emb):
    j = pl.program_id(0)
    width = 10 * emb

    @pl.when(j == 0)
    def _init():
        acc_ref[...] = jnp.zeros_like(acc_ref)

    @pl.when(j < n_edge_blocks)
    def _edge_phase():
        x = x_ref[...]
        v = edv_ref[...]
        r = idx_ref[...][0]
        y = jnp.dot(x, W1_ref[...], preferred_element_type=jnp.float32)
        cols = []
        for a in range(3):
            for b in range(3):
                cols.append((v[:, a:a + 1] * v[:, b:b + 1]) * y)
        cols.append(jnp.ones((be, emb), jnp.float32))
        C = jnp.concatenate(cols, axis=1)

        prev = jnp.concatenate([r[:, :1], r[:, :-1]], axis=1)
        e_iota = jax.lax.broadcasted_iota(jnp.int32, (1, be), 1)
        bnd = jnp.logical_or(e_iota == 0, r != prev).astype(jnp.float32)
        tri = (jax.lax.broadcasted_iota(jnp.int32, (be, be), 0)
               <= jax.lax.broadcasted_iota(jnp.int32, (be, be), 1)
               ).astype(jnp.float32)
        seg = jnp.dot(bnd, tri, preferred_element_type=jnp.float32) - 1.0
        nseg = (seg[0, be - 1] + 1.0).astype(jnp.int32)
        s_iota = jax.lax.broadcasted_iota(
            jnp.int32, (be, be), 0).astype(jnp.float32)
        eq = seg == s_iota
        node_s = jnp.max(jnp.where(eq, r, -1), axis=1)

        pref_ref[...] = jnp.dot(eq.astype(jnp.float32), C,
                                preferred_element_type=jnp.float32)
        node_ref[...] = jnp.broadcast_to(node_s[:, None], (be, 128))

        def body(s, carry):
            node = node_ref[pl.ds(s, 1), :][0, 0]
            upd = acc_ref[pl.ds(node, 1), :] + pref_ref[pl.ds(s, 1), :]
            acc_ref[pl.ds(node, 1), :] = upd
            return carry

        jax.lax.fori_loop(0, nseg, body, 0)

    @pl.when(j >= n_edge_blocks)
    def _node_phase():
        i = j - n_edge_blocks

        @pl.when(i == 0)
        def _ginit():
            gacc_ref[...] = jnp.zeros_like(gacc_ref)

        rows = acc_ref[pl.ds(i * bn, bn), :]
        cnt = jnp.maximum(rows[:, 9 * emb:9 * emb + 1], 1.0)
        outs = []
        for k in range(9):
            a = rows[:, k * emb:(k + 1) * emb] / cnt + b1_ref[...]
            h = a * jax.nn.sigmoid(a)
            outs.append(jnp.sum(h * W2t_ref[...], axis=1, keepdims=True))
        nout = jnp.concatenate(outs, axis=1) + b2_ref[0, 0]
        bat = bat_ref[...][0]
        g_iota = jax.lax.broadcasted_iota(jnp.int32, (_N_GRAPHS, bn), 0)
        G = (bat == g_iota).astype(jnp.float32)
        ext = jnp.concatenate(
            [nout, jnp.ones((bn, 1), jnp.float32),
             jnp.zeros((bn, 128 - 10), jnp.float32)], axis=1)
        gacc_ref[...] = gacc_ref[...] + jnp.dot(
            G, ext, preferred_element_type=jnp.float32)

        @pl.when(i == n_node_blocks - 1)
        def _finish():
            g = gacc_ref[...]
            out_ref[...] = g[:, :9] / jnp.maximum(g[:, 9:10], 1.0)


def kernel(edge_distance_vec, x_edge, edge_index, batch, W1, b1, W2, b2):
    E, emb = x_edge.shape
    N = batch.shape[0]
    be = _pick_block(E, [256, 128, 64, 32, 16, 8])
    bn = _pick_block(N, [400, 200, 100, 80, 40, 16, 8])
    nJ = E // be
    nNB = N // bn
    width = 10 * emb

    idx3 = edge_index.astype(jnp.int32).reshape(nJ, 1, be)
    bat3 = batch.astype(jnp.int32).reshape(nNB, 1, bn)
    b1r = b1.reshape(1, emb).astype(jnp.float32)
    W2t = W2.reshape(emb)[None, :].astype(jnp.float32)
    b2r = b2.reshape(1, 1).astype(jnp.float32)

    grid = (nJ + nNB,)
    ej = lambda j: (jnp.minimum(j, nJ - 1), 0)
    ej3 = lambda j: (jnp.minimum(j, nJ - 1), 0, 0)
    nj3 = lambda j: (jnp.clip(j - nJ, 0, nNB - 1), 0, 0)

    import functools
    kfn = functools.partial(_r2_kernel, n_edge_blocks=nJ, n_node_blocks=nNB,
                            be=be, bn=bn, emb=emb)
    out = pl.pallas_call(
        kfn,
        grid=grid,
        in_specs=[
            pl.BlockSpec((1, 1, be), ej3),
            pl.BlockSpec((be, emb), ej),
            pl.BlockSpec((be, 3), ej),
            pl.BlockSpec((1, 1, bn), nj3),
            pl.BlockSpec((emb, emb), lambda j: (0, 0)),
            pl.BlockSpec((1, emb), lambda j: (0, 0)),
            pl.BlockSpec((1, emb), lambda j: (0, 0)),
            pl.BlockSpec((1, 1), lambda j: (0, 0)),
        ],
        out_specs=pl.BlockSpec((_N_GRAPHS, 9), lambda j: (0, 0)),
        out_shape=jax.ShapeDtypeStruct((_N_GRAPHS, 9), jnp.float32),
        scratch_shapes=[
            pltpu.VMEM((N, width), jnp.float32),
            pltpu.VMEM((be, width), jnp.float32),
            pltpu.VMEM((be, 128), jnp.int32),
            pltpu.VMEM((_N_GRAPHS, 128), jnp.float32),
        ],
        compiler_params=pltpu.CompilerParams(
            dimension_semantics=("arbitrary",),
            vmem_limit_bytes=110 * 1024 * 1024,
        ),
    )(idx3, x_edge.astype(jnp.float32), edge_distance_vec.astype(jnp.float32),
      bat3, W1.astype(jnp.float32), b1r, W2t, b2r)
    return out
